# baseline (speedup 1.0000x reference)
import functools

import jax
import jax.numpy as jnp
from jax import lax
from jax.experimental import pallas as pl
from jax.experimental.pallas import tpu as pltpu
from jax.experimental.pallas import tpu_sc as plsc

BATCH = 16384
HIST = 50
EMBED_DIM = 32
TOTAL = BATCH * HIST
NUM_WORKERS = 32
PER_WORKER = TOTAL // NUM_WORKERS
SUPER = 128
TABLE_SUPER = 1000000 * EMBED_DIM // SUPER
CHUNK = 128
NUM_CHUNKS = PER_WORKER // CHUNK
GROUPS = CHUNK // 16
OUT_COLS = 128
OUT_ROWS = TOTAL * EMBED_DIM // OUT_COLS
OUT_PER_CHUNK = CHUNK * EMBED_DIM // OUT_COLS

_mesh = plsc.VectorSubcoreMesh(core_axis_name="c", subcore_axis_name="s")


@functools.partial(
    pl.kernel,
    out_type=jax.ShapeDtypeStruct((OUT_ROWS, OUT_COLS), jnp.float32),
    mesh=_mesh,
    scratch_types=[
        pltpu.VMEM((2, CHUNK), jnp.int32),
        pltpu.VMEM((2, CHUNK), jnp.int32),
        pltpu.VMEM((2, CHUNK), jnp.int32),
        pltpu.VMEM((2, CHUNK, SUPER), jnp.float32),
        pltpu.VMEM((2, OUT_PER_CHUNK, OUT_COLS), jnp.float32),
        pltpu.SemaphoreType.DMA,
        pltpu.SemaphoreType.DMA,
        pltpu.SemaphoreType.DMA,
    ],
)
def _emb_lookup(idx_hbm, w128_hbm, out_hbm, idxc_v, sidx_v, offv_v, super_v,
                out_v, isem, gsem, wsem):
    cid = lax.axis_index("c")
    sid = lax.axis_index("s")
    wid = sid * 2 + cid
    obase = wid * (PER_WORKER * EMBED_DIM // OUT_COLS)

    def start_idx(c, b):
        pltpu.async_copy(idx_hbm.at[wid, pl.ds(c * CHUNK, CHUNK)],
                         idxc_v.at[b], isem)

    def wait_idx(b):
        pltpu.make_async_copy(idx_hbm.at[wid, pl.ds(0, CHUNK)],
                              idxc_v.at[b], isem).wait()

    def prep_and_fire(b):
        for g in range(GROUPS):
            v = idxc_v[b, pl.ds(g * 16, 16)]
            sidx_v[b, pl.ds(g * 16, 16)] = lax.shift_right_logical(v, 2)
            offv_v[b, pl.ds(g * 16, 16)] = (v & 3) * EMBED_DIM
        pltpu.async_copy(w128_hbm.at[sidx_v.at[b]], super_v.at[b], gsem)

    def wait_gather(b):
        pltpu.make_async_copy(w128_hbm.at[pl.ds(0, CHUNK)],
                              super_v.at[b], gsem).wait()

    def wait_write(b):
        pltpu.make_async_copy(out_v.at[b],
                              out_hbm.at[pl.ds(0, OUT_PER_CHUNK)],
                              wsem).wait()

    def extract(b):
        def group_body(g2, carry):
            offv = offv_v[b, pl.ds(g2 * 16, 16)]
            for k2 in range(16):
                off = offv[k2]
                row = g2 * 16 + k2
                orow = g2 * 4 + k2 // 4
                ocol = (k2 % 4) * EMBED_DIM
                out_v[b, orow, pl.ds(ocol, 16)] = (
                    super_v[b, row, pl.ds(off, 16)])
                out_v[b, orow, pl.ds(ocol + 16, 16)] = (
                    super_v[b, row, pl.ds(off + 16, 16)])
            return carry
        lax.fori_loop(0, GROUPS, group_body, 0)

    def start_write(c, b):
        pltpu.async_copy(
            out_v.at[b],
            out_hbm.at[pl.ds(obase + c * OUT_PER_CHUNK, OUT_PER_CHUNK)],
            wsem)

    start_idx(0, 0)
    start_idx(1, 1)
    wait_idx(0)
    prep_and_fire(0)

    def pair_body(p, carry):
        for sub in range(2):
            c = 2 * p + sub
            b = sub
            nb = 1 - sub

            @pl.when(c + 1 < NUM_CHUNKS)
            def _():
                wait_idx(nb)
                prep_and_fire(nb)

            @pl.when(c + 2 < NUM_CHUNKS)
            def _():
                start_idx(c + 2, b)

            wait_gather(b)

            @pl.when(c >= 2)
            def _():
                wait_write(b)

            extract(b)
            start_write(c, b)
        return carry

    lax.fori_loop(0, NUM_CHUNKS // 2, pair_body, 0)

    wait_write(0)
    wait_write(1)


def kernel(inputs, weight):
    idx = inputs.reshape(NUM_WORKERS, PER_WORKER).astype(jnp.int32)
    w128 = jnp.maximum(weight.reshape(TABLE_SUPER, SUPER),
                       jnp.float32(-3.4e38))
    out = _emb_lookup(idx, w128)
    return out.reshape(BATCH, HIST, EMBED_DIM)


# speedup vs baseline: 1.0090x; 1.0090x over previous
import functools

import jax
import jax.numpy as jnp
from jax import lax
from jax.experimental import pallas as pl
from jax.experimental.pallas import tpu as pltpu
from jax.experimental.pallas import tpu_sc as plsc

BATCH = 16384
HIST = 50
EMBED_DIM = 32
TOTAL = BATCH * HIST
NUM_WORKERS = 32
PER_WORKER = TOTAL // NUM_WORKERS
SUPER = 128
TABLE_SUPER = 1000000 * EMBED_DIM // SUPER
CHUNK = 128
NUM_CHUNKS = PER_WORKER // CHUNK
GROUPS = CHUNK // 16
OUT_COLS = 128
OUT_ROWS = TOTAL * EMBED_DIM // OUT_COLS
OUT_PER_CHUNK = CHUNK * EMBED_DIM // OUT_COLS

_mesh = plsc.VectorSubcoreMesh(core_axis_name="c", subcore_axis_name="s")


@functools.partial(
    pl.kernel,
    out_type=jax.ShapeDtypeStruct((OUT_ROWS, OUT_COLS), jnp.float32),
    mesh=_mesh,
    scratch_types=[
        pltpu.VMEM((2, CHUNK), jnp.int32),
        pltpu.VMEM((2, CHUNK), jnp.int32),
        pltpu.VMEM((2, CHUNK), jnp.int32),
        pltpu.VMEM((2, CHUNK, SUPER), jnp.float32),
        pltpu.VMEM((2, OUT_PER_CHUNK, OUT_COLS), jnp.float32),
        pltpu.SemaphoreType.DMA,
        pltpu.SemaphoreType.DMA,
        pltpu.SemaphoreType.DMA,
    ],
)
def _emb_lookup(idx_hbm, w128_hbm, out_hbm, idxc_v, sidx_v, offv_v, super_v,
                out_v, isem, gsem, wsem):
    cid = lax.axis_index("c")
    sid = lax.axis_index("s")
    wid = sid * 2 + cid
    obase = wid * (PER_WORKER * EMBED_DIM // OUT_COLS)

    def start_idx(c, b):
        pltpu.async_copy(idx_hbm.at[wid, pl.ds(c * CHUNK, CHUNK)],
                         idxc_v.at[b], isem)

    def wait_idx(b):
        pltpu.make_async_copy(idx_hbm.at[wid, pl.ds(0, CHUNK)],
                              idxc_v.at[b], isem).wait()

    def prep_and_fire(b):
        for g in range(GROUPS):
            v = idxc_v[b, pl.ds(g * 16, 16)]
            sidx_v[b, pl.ds(g * 16, 16)] = lax.shift_right_logical(v, 2)
            offv_v[b, pl.ds(g * 16, 16)] = (v & 3) * EMBED_DIM
        pltpu.async_copy(w128_hbm.at[sidx_v.at[b]], super_v.at[b], gsem)

    def wait_gather(b):
        pltpu.make_async_copy(w128_hbm.at[pl.ds(0, CHUNK)],
                              super_v.at[b], gsem).wait()

    def wait_write(b):
        pltpu.make_async_copy(out_v.at[b],
                              out_hbm.at[pl.ds(0, OUT_PER_CHUNK)],
                              wsem).wait()

    def extract(b):
        def group_body(g2, carry):
            offv = offv_v[b, pl.ds(g2 * 16, 16)]
            for k2 in range(16):
                off = offv[k2]
                row = g2 * 16 + k2
                orow = g2 * 4 + k2 // 4
                ocol = (k2 % 4) * EMBED_DIM
                out_v[b, orow, pl.ds(ocol, 16)] = (
                    super_v[b, row, pl.ds(off, 16)])
                out_v[b, orow, pl.ds(ocol + 16, 16)] = (
                    super_v[b, row, pl.ds(off + 16, 16)])
            return carry
        lax.fori_loop(0, GROUPS, group_body, 0)

    def start_write(c, b):
        pltpu.async_copy(
            out_v.at[b],
            out_hbm.at[pl.ds(obase + c * OUT_PER_CHUNK, OUT_PER_CHUNK)],
            wsem)

    start_idx(0, 0)
    start_idx(1, 1)
    wait_idx(0)
    prep_and_fire(0)

    def pair_body(p, carry):
        for sub in range(2):
            c = 2 * p + sub
            b = sub
            nb = 1 - sub

            @pl.when(c + 1 < NUM_CHUNKS)
            def _():
                wait_idx(nb)
                prep_and_fire(nb)

            @pl.when(c + 2 < NUM_CHUNKS)
            def _():
                start_idx(c + 2, b)

            wait_gather(b)

            @pl.when(c >= 2)
            def _():
                wait_write(b)

            extract(b)
            start_write(c, b)
        return carry

    lax.fori_loop(0, NUM_CHUNKS // 2, pair_body, 0)

    wait_write(0)
    wait_write(1)


def kernel(inputs, weight):
    idx = inputs.reshape(NUM_WORKERS, PER_WORKER).astype(jnp.int32)
    w128 = weight.reshape(TABLE_SUPER, SUPER)
    out = _emb_lookup(idx, w128)
    return jnp.maximum(out.reshape(BATCH, HIST, EMBED_DIM),
                       jnp.float32(-3.4e38))


# speedup vs baseline: 1.0633x; 1.0538x over previous
import functools

import jax
import jax.numpy as jnp
from jax import lax
from jax.experimental import pallas as pl
from jax.experimental.pallas import tpu as pltpu
from jax.experimental.pallas import tpu_sc as plsc

BATCH = 16384
HIST = 50
EMBED_DIM = 32
TOTAL = BATCH * HIST
NUM_WORKERS = 32
PER_WORKER = TOTAL // NUM_WORKERS
SUPER = 128
TABLE_SUPER = 1000000 * EMBED_DIM // SUPER
CHUNK = 128
NUM_CHUNKS = PER_WORKER // CHUNK
GROUPS = CHUNK // 16
OUT_COLS = 128
OUT_ROWS = TOTAL * EMBED_DIM // OUT_COLS
OUT_PER_CHUNK = CHUNK * EMBED_DIM // OUT_COLS

_mesh = plsc.VectorSubcoreMesh(core_axis_name="c", subcore_axis_name="s")


@functools.partial(
    pl.kernel,
    out_type=jax.ShapeDtypeStruct((OUT_ROWS, OUT_COLS), jnp.float32),
    mesh=_mesh,
    scratch_types=[
        pltpu.VMEM((2, CHUNK), jnp.int32),
        pltpu.VMEM((2, CHUNK), jnp.int32),
        pltpu.VMEM((2, CHUNK), jnp.int32),
        pltpu.VMEM((2, CHUNK, SUPER), jnp.float32),
        pltpu.VMEM((2, OUT_PER_CHUNK, OUT_COLS), jnp.float32),
        pltpu.SemaphoreType.DMA,
        pltpu.SemaphoreType.DMA,
        pltpu.SemaphoreType.DMA,
    ],
)
def _emb_lookup(idx_hbm, w128_hbm, out_hbm, idxc_v, sidx_v, offv_v, super_v,
                out_v, isem, gsem, wsem):
    cid = lax.axis_index("c")
    sid = lax.axis_index("s")
    wid = sid * 2 + cid
    obase = wid * (PER_WORKER * EMBED_DIM // OUT_COLS)

    def start_idx(c, b):
        pltpu.async_copy(idx_hbm.at[wid, pl.ds(c * CHUNK, CHUNK)],
                         idxc_v.at[b], isem)

    def wait_idx(b):
        pltpu.make_async_copy(idx_hbm.at[wid, pl.ds(0, CHUNK)],
                              idxc_v.at[b], isem).wait()

    def prep_and_fire(b):
        for g in range(GROUPS):
            v = idxc_v[b, pl.ds(g * 16, 16)]
            sidx_v[b, pl.ds(g * 16, 16)] = lax.shift_right_logical(v, 2)
            offv_v[b, pl.ds(g * 16, 16)] = (v & 3) * EMBED_DIM
        pltpu.async_copy(w128_hbm.at[sidx_v.at[b]], super_v.at[b], gsem)

    def wait_gather(b):
        pltpu.make_async_copy(w128_hbm.at[pl.ds(0, CHUNK)],
                              super_v.at[b], gsem).wait()

    def wait_write(b):
        pltpu.make_async_copy(out_v.at[b],
                              out_hbm.at[pl.ds(0, OUT_PER_CHUNK)],
                              wsem).wait()

    def extract(b):
        def group_body(g2, carry):
            offv = offv_v[b, pl.ds(g2 * 16, 16)]
            for k2 in range(16):
                off = offv[k2]
                row = g2 * 16 + k2
                orow = g2 * 4 + k2 // 4
                ocol = (k2 % 4) * EMBED_DIM
                out_v[b, orow, pl.ds(ocol, 16)] = (
                    super_v[b, row, pl.ds(off, 16)])
                out_v[b, orow, pl.ds(ocol + 16, 16)] = (
                    super_v[b, row, pl.ds(off + 16, 16)])
            return carry
        lax.fori_loop(0, GROUPS, group_body, 0)

    def start_write(c, b):
        pltpu.async_copy(
            out_v.at[b],
            out_hbm.at[pl.ds(obase + c * OUT_PER_CHUNK, OUT_PER_CHUNK)],
            wsem)

    start_idx(0, 0)
    start_idx(1, 1)
    wait_idx(0)
    prep_and_fire(0)

    def pair_body(p, carry):
        for sub in range(2):
            c = 2 * p + sub
            b = sub
            nb = 1 - sub

            @pl.when(c + 1 < NUM_CHUNKS)
            def _():
                wait_idx(nb)
                prep_and_fire(nb)

            @pl.when(c + 2 < NUM_CHUNKS)
            def _():
                start_idx(c + 2, b)

            wait_gather(b)

            @pl.when(c >= 2)
            def _():
                wait_write(b)

            extract(b)
            start_write(c, b)
        return carry

    lax.fori_loop(0, NUM_CHUNKS // 2, pair_body, 0)

    wait_write(0)
    wait_write(1)


def kernel(inputs, weight):
    idx = inputs.reshape(NUM_WORKERS, PER_WORKER).astype(jnp.int32)
    w128 = weight.reshape(TABLE_SUPER, SUPER)
    out = _emb_lookup(idx, w128)
    return out.reshape(BATCH, HIST, EMBED_DIM)
